# baseline (speedup 1.0000x reference)
import functools

import jax
import jax.numpy as jnp
from jax import lax
from jax.experimental import pallas as pl
from jax.experimental.pallas import tpu as pltpu
from jax.experimental.pallas import tpu_sc as plsc

NUM_CLASSES = 100000
HIDDEN = 64
BATCH = 16384

NC = 2
NS = 16
L = 16
NW = NC * NS
CHUNK = BATCH // NW
GSZ = 128
NG = CHUNK // GSZ


def _sc_body(labels_hbm, fd_hbm, table_hbm, out_hbm, lbl_v, fd_v, rows_v, sem):
    wid = lax.axis_index("s") * NC + lax.axis_index("c")
    base = wid * CHUNK

    pltpu.sync_copy(labels_hbm.at[pl.ds(base, CHUNK)], lbl_v)
    pltpu.sync_copy(fd_hbm.at[pl.ds(base, CHUNK)], fd_v)

    def mask_step(i, carry):
        sl = pl.ds(i * L, L)
        lbl = lbl_v[sl]
        fd = fd_v[sl]
        lbl_v[sl] = jnp.where(fd == 1, jnp.int32(NUM_CLASSES), lbl)
        return carry

    lax.fori_loop(0, CHUNK // L, mask_step, 0)

    copies = []
    for g in range(NG):
        copies.append(pltpu.async_copy(
            table_hbm.at[lbl_v.at[pl.ds(g * GSZ, GSZ)]],
            rows_v.at[pl.ds(g * GSZ, GSZ), :],
            sem))
    for c in copies:
        c.wait()

    pltpu.sync_copy(rows_v, out_hbm.at[pl.ds(base, CHUNK)])


@functools.lru_cache(maxsize=1)
def _build():
    mesh = plsc.VectorSubcoreMesh(
        core_axis_name="c", subcore_axis_name="s",
        num_cores=NC, num_subcores=NS)
    return pl.kernel(
        _sc_body,
        out_type=jax.ShapeDtypeStruct((BATCH, HIDDEN), jnp.float32),
        mesh=mesh,
        scratch_types=[
            pltpu.VMEM((CHUNK,), jnp.int32),
            pltpu.VMEM((CHUNK,), jnp.int32),
            pltpu.VMEM((CHUNK, HIDDEN), jnp.float32),
            pltpu.SemaphoreType.DMA,
        ],
        compiler_params=pltpu.CompilerParams(use_tc_tiling_on_sc=False),
    )


def kernel(labels, train, force_drop_ids, embedding_table):
    del train
    return _build()(labels, force_drop_ids, embedding_table)


# speedup vs baseline: 1.5043x; 1.5043x over previous
import functools

import jax
import jax.numpy as jnp
from jax import lax
from jax.experimental import pallas as pl
from jax.experimental.pallas import tpu as pltpu
from jax.experimental.pallas import tpu_sc as plsc

NUM_CLASSES = 100000
HIDDEN = 64
BATCH = 16384

NC = 2
NS = 16
L = 16
NW = NC * NS
CHUNK = BATCH // NW


def _sc_body(labels_hbm, fd_hbm, table_hbm, out_hbm, lbl_v, fd_v, rows_v, sem):
    wid = lax.axis_index("s") * NC + lax.axis_index("c")
    base = wid * CHUNK

    pltpu.sync_copy(labels_hbm.at[pl.ds(base, CHUNK)], lbl_v)
    pltpu.sync_copy(fd_hbm.at[pl.ds(base, CHUNK)], fd_v)

    def fire_group(g, carry):
        sl = pl.ds(g * L, L)
        lbl = lbl_v[sl]
        fd = fd_v[sl]
        idx = jnp.where(fd == 1, jnp.int32(NUM_CLASSES), lbl)
        for j in range(L):
            pltpu.async_copy(table_hbm.at[idx[j]], rows_v.at[g * L + j], sem)
        return carry

    lax.fori_loop(0, CHUNK // L, fire_group, 0)

    pltpu.make_async_copy(table_hbm.at[pl.ds(0, CHUNK)], rows_v, sem).wait()

    pltpu.sync_copy(rows_v, out_hbm.at[pl.ds(base, CHUNK)])


@functools.lru_cache(maxsize=1)
def _build():
    mesh = plsc.VectorSubcoreMesh(
        core_axis_name="c", subcore_axis_name="s",
        num_cores=NC, num_subcores=NS)
    return pl.kernel(
        _sc_body,
        out_type=jax.ShapeDtypeStruct((BATCH, HIDDEN), jnp.float32),
        mesh=mesh,
        scratch_types=[
            pltpu.VMEM((CHUNK,), jnp.int32),
            pltpu.VMEM((CHUNK,), jnp.int32),
            pltpu.VMEM((CHUNK, HIDDEN), jnp.float32),
            pltpu.SemaphoreType.DMA,
        ],
        compiler_params=pltpu.CompilerParams(use_tc_tiling_on_sc=True),
    )


def kernel(labels, train, force_drop_ids, embedding_table):
    del train
    return _build()(labels, force_drop_ids, embedding_table)
